# baseline (speedup 1.0000x reference)
import functools
import math

import jax
import jax.numpy as jnp
import numpy as np
from jax import lax
from jax.experimental import pallas as pl
from jax.experimental.pallas import tpu as pltpu
from jax.experimental.pallas import tpu_sc as plsc

B, N, K = 8, 2048, 32
CIN, COUT, H = 3, 64, 8
DEPTH = COUT // H
TILE = 256
CH = 16
NCH = N // CH
ATT_TILE = 32
GATHER_WIN = 256
PB = 16
OB = 64


def _qkv_body(xt_ref, w_ref, q_ref, kv_ref):
    xt = xt_ref[...]
    w = w_ref[...]
    qkv = jnp.dot(xt, w, preferred_element_type=jnp.float32)
    q_ref[...] = qkv[:, :COUT]
    kv_ref[...] = qkv[:, COUT:]


def _project_qkv(xt_flat, w_cat):
    return pl.pallas_call(
        _qkv_body,
        out_shape=(
            jax.ShapeDtypeStruct((B * N, COUT), jnp.float32),
            jax.ShapeDtypeStruct((B * N, 2 * COUT), jnp.float32),
        ),
    )(xt_flat, w_cat)


def _knn_chunk_body(cb_ref, ctile_ref, dist_ref, cid_ref, s_ref):
    cb = cb_ref[0]
    ctile = ctile_ref[0]
    sq_all = jnp.sum(cb * cb, axis=0, keepdims=True)
    sq_t = jnp.sum(ctile * ctile, axis=1, keepdims=True)
    g = jnp.dot(ctile, cb, preferred_element_type=jnp.float32)
    dist = sq_t + sq_all - 2.0 * g
    dist_ref[0] = dist
    cmin = jnp.min(dist.reshape(TILE, NCH, CH), axis=2)
    ct = cmin.T
    iota = lax.broadcasted_iota(jnp.int32, (NCH, TILE), 0)
    big = jnp.float32(np.inf)
    for j in range(K):
        v = jnp.min(ct, axis=0, keepdims=True)
        cand = jnp.where(ct == v, iota, jnp.int32(NCH))
        pos = jnp.min(cand, axis=0, keepdims=True)
        cid_ref[0, j, :] = pos[0]
        ct = jnp.where(iota == pos, big, ct)
        if j == K - 1:
            s_ref[0, 0, :] = v[0]


def _knn_chunks(coordinate, coord_t):
    grid = (B, N // TILE)
    return pl.pallas_call(
        _knn_chunk_body,
        grid=grid,
        in_specs=[
            pl.BlockSpec((1, CIN, N), lambda b, t: (b, 0, 0)),
            pl.BlockSpec((1, TILE, CIN), lambda b, t: (b, t, 0)),
        ],
        out_specs=[
            pl.BlockSpec((1, TILE, N), lambda b, t: (b, t, 0)),
            pl.BlockSpec((1, K, TILE), lambda b, t: (b, 0, t)),
            pl.BlockSpec((1, 1, TILE), lambda b, t: (b, 0, t)),
        ],
        out_shape=[
            jax.ShapeDtypeStruct((B, N, N), jnp.float32),
            jax.ShapeDtypeStruct((B, K, N), jnp.int32),
            jax.ShapeDtypeStruct((B, 1, N), jnp.float32),
        ],
    )(coordinate, coord_t)


def _sc_select(d_tab, crow_flat, s16):
    info = plsc.get_sparse_core_info()
    nw = info.num_cores * info.num_subcores
    pts = B * N
    per_w = pts // nw
    nblk = per_w // PB
    mesh = plsc.VectorSubcoreMesh(core_axis_name="c", subcore_axis_name="s")
    inf32 = jnp.float32(np.inf)
    cp = pltpu.CompilerParams(needs_layout_passes=False,
                              use_tc_tiling_on_sc=False)

    @functools.partial(
        pl.kernel,
        mesh=mesh,
        compiler_params=cp,
        out_type=jax.ShapeDtypeStruct((pts * OB,), jnp.int32),
        scratch_types=[
            pltpu.VMEM((2, PB * K), jnp.int32),
            pltpu.VMEM((2, PB * K, CH), jnp.float32),
            pltpu.VMEM((2, PB, 16), jnp.float32),
            pltpu.VMEM((PB * OB,), jnp.int32),
            pltpu.SemaphoreType.DMA,
            pltpu.SemaphoreType.DMA,
            pltpu.SemaphoreType.DMA,
            pltpu.SemaphoreType.DMA,
        ],
    )
    def select_kernel(d_hbm, c_hbm, s_hbm, o_hbm, cid_v, cand_v, s_v, ob_v,
                      gsem0, gsem1, psem, osem):
        wid = lax.axis_index("s") * info.num_cores + lax.axis_index("c")
        base = wid * per_w
        gsems = (gsem0, gsem1)

        def fetch(blk, buf):
            p0 = base + blk * PB
            pltpu.async_copy(c_hbm.at[pl.ds(p0 * K, PB * K)],
                             cid_v.at[buf], psem).wait()
            pltpu.async_copy(s_hbm.at[pl.ds(p0, PB)], s_v.at[buf],
                             psem).wait()
            for q in range(PB * K // 128):
                pltpu.async_copy(
                    d_hbm.at[cid_v.at[buf, pl.ds(q * 128, 128)]],
                    cand_v.at[buf, pl.ds(q * 128, 128)], gsems[buf])

        def drain(buf):
            for q in range(PB * K // 128):
                pltpu.make_async_copy(
                    d_hbm.at[cid_v.at[buf, pl.ds(q * 128, 128)]],
                    cand_v.at[buf, pl.ds(q * 128, 128)], gsems[buf]).wait()

        fetch(0, 0)
        lane = lax.iota(jnp.int32, 16)

        def process(blk, buf):

            @pl.when(blk + 1 < nblk)
            def _():
                fetch(blk + 1, 1 - buf)

            drain(buf)

            @pl.loop(0, PB)
            def _(pi):
                p = base + blk * PB + pi
                svec = s_v[buf, pi]

                def scan_body(j, carry):
                    t0, t1 = carry
                    row = cand_v[buf, pi * K + j]

                    def insert(c):
                        a, b = c
                        sv = lax.sort(jnp.where(row <= svec, row, inf32))
                        lo = lax.sort(jnp.minimum(b, lax.rev(sv, (0,))))
                        rlo = lax.rev(lo, (0,))
                        return (lax.sort(jnp.minimum(a, rlo)),
                                lax.sort(jnp.maximum(a, rlo)))

                    return lax.cond(jnp.any(row <= svec), insert,
                                    lambda c: c, (t0, t1))

                t0, t1 = lax.fori_loop(
                    0, K, scan_body,
                    (jnp.full((16,), inf32), jnp.full((16,), inf32)))
                t = lax.reduce_max(t1, (0,))
                tvec = jnp.full((16,), jnp.float32(0)) + t

                fbase = lax.div(p, jnp.int32(N)) * N - p * N
                c0 = cid_v[buf, pl.ds(pi * K, 16)]
                c1 = cid_v[buf, pl.ds(pi * K + 16, 16)]

                def emit_body(j, ptr):
                    row = cand_v[buf, pi * K + j]
                    m2 = row <= tvec

                    def do(ptr):
                        half = jnp.where(j < 16, c0, c1)
                        dn = lax.GatherDimensionNumbers(
                            offset_dims=(), collapsed_slice_dims=(0,),
                            start_index_map=(0,))
                        rvec = lax.gather(
                            half,
                            jnp.full((16, 1), lax.rem(j, 16), jnp.int32),
                            dn, slice_sizes=(1,),
                            mode=lax.GatherScatterMode.PROMISE_IN_BOUNDS)
                        gidx = rvec * CH + lane + fbase
                        plsc.store_compressed(
                            ob_v.at[pl.ds(pi * OB + ptr, 16)], gidx, mask=m2)
                        cnt = jnp.sum(jnp.where(m2, 1, 0))
                        return jnp.minimum(ptr + cnt, jnp.int32(OB - 16))

                    return lax.cond(jnp.any(m2), do, lambda q: q, ptr)

                lax.fori_loop(0, K, emit_body, jnp.int32(0))

            pltpu.async_copy(
                ob_v, o_hbm.at[pl.ds((base + blk * PB) * OB, PB * OB)],
                osem).wait()

        @pl.loop(0, nblk // 2)
        def _(hb):
            process(2 * hb, 0)
            process(2 * hb + 1, 1)

    return select_kernel(d_tab, crow_flat, s16)


def _sc_gather(kv_table, idx_flat):
    total = idx_flat.shape[0]
    idx2d = idx_flat.reshape(1, total)
    mesh = plsc.VectorSubcoreMesh(core_axis_name="c", subcore_axis_name="s")

    @functools.partial(
        pl.kernel,
        mesh=mesh,
        out_type=jax.ShapeDtypeStruct((total, 2 * COUT), jnp.float32),
    )
    def gather_kernel(kv_hbm, i_hbm, o_hbm):
        def body(i_vmem, o_vmem):
            pltpu.sync_copy(kv_hbm.at[i_vmem.at[0]], o_vmem)

        pltpu.emit_pipeline(
            body,
            grid=(total // GATHER_WIN,),
            in_specs=[pl.BlockSpec((1, GATHER_WIN),
                                   index_map=lambda i: (0, i))],
            out_specs=[pl.BlockSpec((GATHER_WIN, 2 * COUT),
                                    index_map=lambda i: (i, 0))],
            core_axis_name=("c", "s"),
            dimension_semantics=(pltpu.PARALLEL,),
        )(i_hbm, o_hbm)

    return gather_kernel(kv_table, idx2d)


def _attn_body(q_ref, kvnb_ref, vself_ref, seg_ref, out_ref):
    q = q_ref[...]
    kvnb = kvnb_ref[...]
    knb = kvnb[:, :COUT].reshape(ATT_TILE, K, COUT)
    vnb = kvnb[:, COUT:].reshape(ATT_TILE, K, COUT)
    prod = knb * q[:, None, :]
    seg = seg_ref[...]
    e = jnp.dot(prod.reshape(ATT_TILE * K, COUT), seg,
                preferred_element_type=jnp.float32).reshape(ATT_TILE, K, COUT)
    m = jnp.max(e, axis=1, keepdims=True)
    ex = jnp.exp(e - m)
    s = jnp.sum(ex, axis=1, keepdims=True)
    a = ex / s
    out = jnp.sum(a * vnb, axis=1)
    out_ref[...] = out - vself_ref[...]


def _attention(q_table, kvnb, v_self):
    grid = (B * N // ATT_TILE,)
    seg = jnp.kron(jnp.eye(H, dtype=jnp.float32),
                   jnp.ones((DEPTH, DEPTH), jnp.float32)) / math.sqrt(DEPTH)
    return pl.pallas_call(
        _attn_body,
        grid=grid,
        in_specs=[
            pl.BlockSpec((ATT_TILE, COUT), lambda t: (t, 0)),
            pl.BlockSpec((ATT_TILE * K, 2 * COUT), lambda t: (t, 0)),
            pl.BlockSpec((ATT_TILE, COUT), lambda t: (t, 0)),
            pl.BlockSpec((COUT, COUT), lambda t: (0, 0)),
        ],
        out_specs=pl.BlockSpec((ATT_TILE, COUT), lambda t: (t, 0)),
        out_shape=jax.ShapeDtypeStruct((B * N, COUT), jnp.float32),
    )(q_table, kvnb, v_self, seg)


def kernel(x, coordinate, Wq, Wk, Wv):
    xt_flat = x.transpose(0, 2, 1).reshape(B * N, CIN)
    coord_t = coordinate.transpose(0, 2, 1)
    w_cat = jnp.concatenate([Wq.T, Wk.T, Wv.T], axis=1)
    q_table, kv_table = _project_qkv(xt_flat, w_cat)
    dist, cid, s = _knn_chunks(coordinate, coord_t)
    point_id = jnp.arange(B * N, dtype=jnp.int32).reshape(B, 1, N)
    crow = (cid + point_id * NCH).transpose(0, 2, 1).reshape(B * N * K)
    s16 = jnp.broadcast_to(s.reshape(B * N, 1), (B * N, 16))
    d_tab = dist.reshape(B * N * NCH, CH)

    nbidx = _sc_select(d_tab, crow, s16).reshape(B * N, OB)[:, :K]
    kvnb = _sc_gather(kv_table, nbidx.reshape(B * N * K))
    v_self = kv_table[:, COUT:]
    out_rows = _attention(q_table, kvnb, v_self)
    return out_rows.reshape(B, N, COUT).transpose(0, 2, 1)
